# Initial kernel scaffold; baseline (speedup 1.0000x reference)
import jax
import jax.numpy as jnp
from jax.experimental import pallas as pl


def kernel(x, edge_index, edge_attr, batch, atom_W, atom_b, edge_W, edge_b, Wf, bf, Ws, bs, gamma, beta, fc_hW, fc_hb, hid_W, hid_b, out_W, out_b):
    raise NotImplementedError("write your pallas kernel here")


# speedup vs baseline: 1.7203x; 1.7203x over previous
import functools

import jax
import jax.numpy as jnp
from jax import lax
from jax.experimental import pallas as pl
from jax.experimental.pallas import tpu as pltpu
from jax.experimental.pallas import tpu_sc as plsc

NC = 2
NS = 16
NW = NC * NS
EB = 80


def _sc_mesh():
    return plsc.VectorSubcoreMesh(
        core_axis_name="c", subcore_axis_name="s", num_cores=NC, num_subcores=NS)


def _mm_bias_body(x_ref, w_ref, b_ref, o_ref):
    o_ref[...] = jnp.dot(x_ref[...], w_ref[...],
                         preferred_element_type=jnp.float32) + b_ref[...]


def _tc_mm_bias(x, w, b, block_rows):
    n, k = x.shape
    m = w.shape[1]
    grid = n // block_rows
    return pl.pallas_call(
        _mm_bias_body,
        grid=(grid,),
        in_specs=[
            pl.BlockSpec((block_rows, k), lambda i: (i, 0)),
            pl.BlockSpec((k, m), lambda i: (0, 0)),
            pl.BlockSpec((1, m), lambda i: (0, 0)),
        ],
        out_specs=pl.BlockSpec((block_rows, m), lambda i: (i, 0)),
        out_shape=jax.ShapeDtypeStruct((n, m), jnp.float32),
    )(x, w, b)


def _edge_const_body(ea_ref, m_ref, c_ref, o0_ref, o1_ref, o2_ref):
    t = jnp.dot(ea_ref[...], m_ref[...],
                preferred_element_type=jnp.float32) + c_ref[...]
    o0_ref[...] = t[:, :256]
    o1_ref[...] = t[:, 256:512]
    o2_ref[...] = t[:, 512:]


def _tc_edge_const(edge_attr, mcat, ccat, block_rows):
    e = edge_attr.shape[0]
    grid = e // block_rows
    out = jax.ShapeDtypeStruct((e, 256), jnp.float32)
    return pl.pallas_call(
        _edge_const_body,
        grid=(grid,),
        in_specs=[
            pl.BlockSpec((block_rows, 16), lambda i: (i, 0)),
            pl.BlockSpec((16, 768), lambda i: (0, 0)),
            pl.BlockSpec((1, 768), lambda i: (0, 0)),
        ],
        out_specs=[pl.BlockSpec((block_rows, 256), lambda i: (i, 0))] * 3,
        out_shape=[out, out, out],
    )(edge_attr, mcat, ccat)


def _proj_body(h_ref, wd_ref, ws_ref, pd_ref, ps_ref):
    h = h_ref[...]
    pd_ref[...] = jnp.dot(h, wd_ref[...], preferred_element_type=jnp.float32)
    ps_ref[...] = jnp.dot(h, ws_ref[...], preferred_element_type=jnp.float32)


def _tc_proj(h, wd, ws, block_rows):
    n = h.shape[0]
    grid = n // block_rows
    out = jax.ShapeDtypeStruct((n, 256), jnp.float32)
    return pl.pallas_call(
        _proj_body,
        grid=(grid,),
        in_specs=[
            pl.BlockSpec((block_rows, 128), lambda i: (i, 0)),
            pl.BlockSpec((128, 256), lambda i: (0, 0)),
            pl.BlockSpec((128, 256), lambda i: (0, 0)),
        ],
        out_specs=[pl.BlockSpec((block_rows, 256), lambda i: (i, 0))] * 2,
        out_shape=[out, out],
    )(h, wd, ws)


def _act_body(t_ref, o_ref):
    t = t_ref[...]
    o_ref[...] = jax.nn.sigmoid(t[:, :128]) * jax.nn.softplus(t[:, 128:])


def _tc_act(t, block_rows):
    e = t.shape[0]
    grid = e // block_rows
    return pl.pallas_call(
        _act_body,
        grid=(grid,),
        in_specs=[pl.BlockSpec((block_rows, 256), lambda i: (i, 0))],
        out_specs=pl.BlockSpec((block_rows, 128), lambda i: (i, 0)),
        out_shape=jax.ShapeDtypeStruct((e, 128), jnp.float32),
    )(t)


def _norm_body(aggp_ref, h_ref, g_ref, b_ref, o_ref):
    agg = aggp_ref[0] + aggp_ref[1]
    m = jnp.mean(agg, axis=0, keepdims=True)
    d = agg - m
    v = jnp.mean(d * d, axis=0, keepdims=True)
    o_ref[...] = d * lax.rsqrt(v + 1e-5) * g_ref[...] + b_ref[...] + h_ref[...]


def _tc_norm(aggp, h, g, b):
    return pl.pallas_call(
        _norm_body,
        out_shape=jax.ShapeDtypeStruct(h.shape, jnp.float32),
    )(aggp, h, g, b)


def _pool_mlp_body(h_ref, batch_ref, fcw_ref, fcb_ref, hw_ref, hb_ref,
                   ow_ref, ob_ref, o_ref):
    n = h_ref.shape[0]
    gidx = lax.broadcasted_iota(jnp.int32, (n, 16), 1)
    oh = (batch_ref[...] == gidx).astype(jnp.float32)
    sums = lax.dot_general(oh, h_ref[...], (((0,), (0,)), ((), ())),
                           preferred_element_type=jnp.float32)
    cnts = jnp.sum(oh, axis=0)
    pooled = sums / jnp.clip(cnts, 1.0)[:, None]
    h2 = jnp.dot(pooled, fcw_ref[...],
                 preferred_element_type=jnp.float32) + fcb_ref[...]
    for l in range(3):
        h2 = jax.nn.relu(jnp.dot(h2, hw_ref[l],
                                 preferred_element_type=jnp.float32) + hb_ref[l])
    out = jnp.dot(h2, ow_ref[...], preferred_element_type=jnp.float32)
    o_ref[...] = out.T + ob_ref[...]


def _tc_pool_mlp(h, batch2d, fc_hW, fc_hb, hid_W, hid_b, out_W, out_b):
    return pl.pallas_call(
        _pool_mlp_body,
        out_shape=jax.ShapeDtypeStruct((1, 16), jnp.float32),
    )(h, batch2d, fc_hW, fc_hb, hid_W, hid_b, out_W, out_b)


def _sc_gather_sum(pd, ps, c, src, dst):
    e = c.shape[0]
    ew = e // NW
    nch = ew // EB

    @functools.partial(
        pl.kernel,
        out_type=jax.ShapeDtypeStruct((e, 256), jnp.float32),
        mesh=_sc_mesh(),
        scratch_types=[
            pltpu.VMEM((EB,), jnp.int32),
            pltpu.VMEM((EB,), jnp.int32),
            pltpu.VMEM((EB, 256), jnp.float32),
            pltpu.VMEM((EB, 256), jnp.float32),
            pltpu.VMEM((EB, 256), jnp.float32),
            pltpu.SemaphoreType.DMA,
            pltpu.SemaphoreType.DMA,
        ],
    )
    def k(pd_h, ps_h, c_h, src_h, dst_h, out_h, idxd, idxs, rd, rs, cc,
          sem_a, sem_b):
        wid = lax.axis_index("c") * NS + lax.axis_index("s")

        def chunk(kk, _):
            base = wid * ew + kk * EB
            pltpu.sync_copy(dst_h.at[pl.ds(base, EB)], idxd)
            pltpu.sync_copy(src_h.at[pl.ds(base, EB)], idxs)
            cp_a = pltpu.async_copy(pd_h.at[idxd], rd, sem_a)
            cp_b = pltpu.async_copy(ps_h.at[idxs], rs, sem_b)
            pltpu.sync_copy(c_h.at[pl.ds(base, EB)], cc)
            cp_a.wait()
            cp_b.wait()

            def ebody(i, _):
                for j in range(16):
                    sl = pl.ds(j * 16, 16)
                    rd[i, sl] = rd[i, sl] + rs[i, sl] + cc[i, sl]
                return 0

            lax.fori_loop(0, EB, ebody, 0)
            pltpu.sync_copy(rd, out_h.at[pl.ds(base, EB)])
            return 0

        lax.fori_loop(0, nch, chunk, 0)

    return k(pd, ps, c, src, dst)


def _sc_scatter_add(msg, dst, zeros):
    e, d = msg.shape
    n = zeros.shape[0]
    ew = e // NW
    nch = ew // EB
    rps = (n // NS) // 8 * 8
    tail = n - NS * rps

    @functools.partial(
        pl.kernel,
        out_type=jax.ShapeDtypeStruct((NC, n, d), jnp.float32),
        mesh=_sc_mesh(),
        scratch_types=[
            pltpu.VMEM((EB,), jnp.int32),
            pltpu.VMEM((EB, d), jnp.float32),
            pltpu.VMEM_SHARED((n, d), jnp.float32),
        ],
    )
    def k(msg_h, dst_h, z_h, out_h, idx, mv, shared):
        cid = lax.axis_index("c")
        sid = lax.axis_index("s")
        wid = cid * NS + sid
        row0 = sid * rps
        pltpu.sync_copy(z_h.at[pl.ds(row0, rps)], shared.at[pl.ds(row0, rps)])

        @pl.when(sid == 0)
        def _():
            pltpu.sync_copy(z_h.at[pl.ds(NS * rps, tail)],
                            shared.at[pl.ds(NS * rps, tail)])

        plsc.subcore_barrier()

        def chunk(kk, _):
            base = wid * ew + kk * EB
            pltpu.sync_copy(dst_h.at[pl.ds(base, EB)], idx)
            pltpu.sync_copy(msg_h.at[pl.ds(base, EB)], mv)
            pltpu.sync_copy(mv, shared.at[idx], add=True)
            return 0

        lax.fori_loop(0, nch, chunk, 0)
        plsc.subcore_barrier()
        pltpu.sync_copy(shared.at[pl.ds(row0, rps)],
                        out_h.at[cid, pl.ds(row0, rps)])

        @pl.when(sid == 0)
        def _():
            pltpu.sync_copy(shared.at[pl.ds(NS * rps, tail)],
                            out_h.at[cid, pl.ds(NS * rps, tail)])

    return k(msg, dst, zeros)


def kernel(x, edge_index, edge_attr, batch, atom_W, atom_b, edge_W, edge_b,
           Wf, bf, Ws, bs, gamma, beta, fc_hW, fc_hb, hid_W, hid_b,
           out_W, out_b):
    n, d = x.shape
    e = edge_attr.shape[0]
    src = edge_index[0]
    dst = edge_index[1]

    wds, wss, ms, cs = [], [], [], []
    for l in range(3):
        wf_i, wf_j, wf_e = Wf[l][:d], Wf[l][d:2 * d], Wf[l][2 * d:]
        ws_i, ws_j, ws_e = Ws[l][:d], Ws[l][d:2 * d], Ws[l][2 * d:]
        wds.append(jnp.concatenate([wf_i, ws_i], axis=1))
        wss.append(jnp.concatenate([wf_j, ws_j], axis=1))
        ms.append(jnp.concatenate([edge_W @ wf_e, edge_W @ ws_e], axis=1))
        cs.append(jnp.concatenate([edge_b @ wf_e + bf[l],
                                   edge_b @ ws_e + bs[l]]))
    mcat = jnp.concatenate(ms, axis=1)
    ccat = jnp.concatenate(cs)[None, :]

    h = _tc_mm_bias(x, atom_W, atom_b[None, :], block_rows=2000)
    c_layers = _tc_edge_const(edge_attr, mcat, ccat, block_rows=4000)
    zeros = jnp.zeros((n, d), dtype=jnp.float32)

    for l in range(3):
        pd, ps = _tc_proj(h, wds[l], wss[l], block_rows=2000)
        t = _sc_gather_sum(pd, ps, c_layers[l], src, dst)
        msg = _tc_act(t, block_rows=4000)
        aggp = _sc_scatter_add(msg, dst, zeros)
        h = _tc_norm(aggp, h, gamma[l][None, :], beta[l][None, :])

    out = _tc_pool_mlp(h, batch[:, None], fc_hW, fc_hb[None, :],
                       hid_W, hid_b, out_W, out_b[None, :])
    return jnp.reshape(out, (16,))


# speedup vs baseline: 2.5026x; 1.4547x over previous
import functools

import jax
import jax.numpy as jnp
from jax import lax
from jax.experimental import pallas as pl
from jax.experimental.pallas import tpu as pltpu
from jax.experimental.pallas import tpu_sc as plsc

NC = 2
NS = 16
NW = NC * NS
EB = 80


def _sc_mesh():
    return plsc.VectorSubcoreMesh(
        core_axis_name="c", subcore_axis_name="s", num_cores=NC, num_subcores=NS)


def _mm_bias_body(x_ref, w_ref, b_ref, o_ref):
    o_ref[...] = jnp.dot(x_ref[...], w_ref[...],
                         preferred_element_type=jnp.float32) + b_ref[...]


def _tc_mm_bias(x, w, b, block_rows):
    n, k = x.shape
    m = w.shape[1]
    grid = n // block_rows
    return pl.pallas_call(
        _mm_bias_body,
        grid=(grid,),
        in_specs=[
            pl.BlockSpec((block_rows, k), lambda i: (i, 0)),
            pl.BlockSpec((k, m), lambda i: (0, 0)),
            pl.BlockSpec((1, m), lambda i: (0, 0)),
        ],
        out_specs=pl.BlockSpec((block_rows, m), lambda i: (i, 0)),
        out_shape=jax.ShapeDtypeStruct((n, m), jnp.float32),
    )(x, w, b)


def _edge_const_body(ea_ref, m_ref, c_ref, o0_ref, o1_ref, o2_ref):
    t = jnp.dot(ea_ref[...], m_ref[...],
                preferred_element_type=jnp.float32) + c_ref[...]
    o0_ref[...] = t[:, :256]
    o1_ref[...] = t[:, 256:512]
    o2_ref[...] = t[:, 512:]


def _tc_edge_const(edge_attr, mcat, ccat, block_rows):
    e = edge_attr.shape[0]
    grid = e // block_rows
    out = jax.ShapeDtypeStruct((e, 256), jnp.float32)
    return pl.pallas_call(
        _edge_const_body,
        grid=(grid,),
        in_specs=[
            pl.BlockSpec((block_rows, 16), lambda i: (i, 0)),
            pl.BlockSpec((16, 768), lambda i: (0, 0)),
            pl.BlockSpec((1, 768), lambda i: (0, 0)),
        ],
        out_specs=[pl.BlockSpec((block_rows, 256), lambda i: (i, 0))] * 3,
        out_shape=[out, out, out],
    )(edge_attr, mcat, ccat)


def _proj_body(h_ref, wd_ref, ws_ref, pd_ref, ps_ref):
    h = h_ref[...]
    pd_ref[...] = jnp.dot(h, wd_ref[...], preferred_element_type=jnp.float32)
    ps_ref[...] = jnp.dot(h, ws_ref[...], preferred_element_type=jnp.float32)


def _tc_proj(h, wd, ws, block_rows):
    n = h.shape[0]
    grid = n // block_rows
    out = jax.ShapeDtypeStruct((n, 256), jnp.float32)
    return pl.pallas_call(
        _proj_body,
        grid=(grid,),
        in_specs=[
            pl.BlockSpec((block_rows, 128), lambda i: (i, 0)),
            pl.BlockSpec((128, 256), lambda i: (0, 0)),
            pl.BlockSpec((128, 256), lambda i: (0, 0)),
        ],
        out_specs=[pl.BlockSpec((block_rows, 256), lambda i: (i, 0))] * 2,
        out_shape=[out, out],
    )(h, wd, ws)


def _act_body(t_ref, c_ref, o_ref):
    t = t_ref[...] + c_ref[...]
    o_ref[...] = jax.nn.sigmoid(t[:, :128]) * jax.nn.softplus(t[:, 128:])


def _tc_act(t, c, block_rows):
    e = t.shape[0]
    grid = e // block_rows
    return pl.pallas_call(
        _act_body,
        grid=(grid,),
        in_specs=[pl.BlockSpec((block_rows, 256), lambda i: (i, 0))] * 2,
        out_specs=pl.BlockSpec((block_rows, 128), lambda i: (i, 0)),
        out_shape=jax.ShapeDtypeStruct((e, 128), jnp.float32),
    )(t, c)


def _norm_body(aggp_ref, h_ref, g_ref, b_ref, o_ref):
    agg = aggp_ref[0] + aggp_ref[1]
    m = jnp.mean(agg, axis=0, keepdims=True)
    d = agg - m
    v = jnp.mean(d * d, axis=0, keepdims=True)
    o_ref[...] = d * lax.rsqrt(v + 1e-5) * g_ref[...] + b_ref[...] + h_ref[...]


def _tc_norm(aggp, h, g, b):
    return pl.pallas_call(
        _norm_body,
        out_shape=jax.ShapeDtypeStruct(h.shape, jnp.float32),
    )(aggp, h, g, b)


def _pool_mlp_body(h_ref, batch_ref, fcw_ref, fcb_ref, hw_ref, hb_ref,
                   ow_ref, ob_ref, o_ref):
    n = h_ref.shape[0]
    gidx = lax.broadcasted_iota(jnp.int32, (n, 16), 1)
    oh = (batch_ref[...] == gidx).astype(jnp.float32)
    sums = lax.dot_general(oh, h_ref[...], (((0,), (0,)), ((), ())),
                           preferred_element_type=jnp.float32)
    cnts = jnp.sum(oh, axis=0)
    pooled = sums / jnp.clip(cnts, 1.0)[:, None]
    h2 = jnp.dot(pooled, fcw_ref[...],
                 preferred_element_type=jnp.float32) + fcb_ref[...]
    for l in range(3):
        h2 = jax.nn.relu(jnp.dot(h2, hw_ref[l],
                                 preferred_element_type=jnp.float32) + hb_ref[l])
    out = jnp.dot(h2, ow_ref[...], preferred_element_type=jnp.float32)
    o_ref[...] = out.T + ob_ref[...]


def _tc_pool_mlp(h, batch2d, fc_hW, fc_hb, hid_W, hid_b, out_W, out_b):
    return pl.pallas_call(
        _pool_mlp_body,
        out_shape=jax.ShapeDtypeStruct((1, 16), jnp.float32),
    )(h, batch2d, fc_hW, fc_hb, hid_W, hid_b, out_W, out_b)


def _sc_gather_sum(pd, ps, src, dst):
    n = pd.shape[0]
    e = src.shape[0]
    ew = e // NW
    nch = ew // EB
    npair = (nch + 1) // 2

    @functools.partial(
        pl.kernel,
        out_type=jax.ShapeDtypeStruct((e, 256), jnp.float32),
        mesh=_sc_mesh(),
        scratch_types=[
            pltpu.VMEM((ew,), jnp.int32),
            pltpu.VMEM((ew,), jnp.int32),
            pltpu.VMEM((2, EB, 256), jnp.float32),
            pltpu.VMEM((2, EB, 256), jnp.float32),
            pltpu.SemaphoreType.DMA,
            pltpu.SemaphoreType.DMA,
            pltpu.SemaphoreType.DMA,
            pltpu.SemaphoreType.DMA,
        ],
    )
    def k(pd_h, ps_h, src_h, dst_h, out_h, idxd, idxs, rd, rs,
          sd0, sd1, ss0, ss1):
        wid = lax.axis_index("c") * NS + lax.axis_index("s")
        base_w = wid * ew
        sems_d = (sd0, sd1)
        sems_s = (ss0, ss1)
        pltpu.sync_copy(dst_h.at[pl.ds(base_w, ew)], idxd)
        pltpu.sync_copy(src_h.at[pl.ds(base_w, ew)], idxs)

        def issue(kk, b):
            isl = pl.ds(kk * EB, EB)
            pltpu.async_copy(pd_h.at[idxd.at[isl]], rd.at[b], sems_d[b])
            pltpu.async_copy(ps_h.at[idxs.at[isl]], rs.at[b], sems_s[b])

        issue(0, 0)
        issue(1, 1)

        def pair(p, _):
            for b in range(2):
                kk = 2 * p + b

                @pl.when(kk < nch)
                def _():
                    pltpu.make_async_copy(pd_h.at[idxd.at[pl.ds(0, EB)]],
                                          rd.at[b], sems_d[b]).wait()
                    pltpu.make_async_copy(ps_h.at[idxs.at[pl.ds(0, EB)]],
                                          rs.at[b], sems_s[b]).wait()

                    def ebody(i, _):
                        for j in range(16):
                            sl = pl.ds(j * 16, 16)
                            rd[b, i, sl] = rd[b, i, sl] + rs[b, i, sl]
                        return 0

                    lax.fori_loop(0, EB, ebody, 0)
                    pltpu.sync_copy(rd.at[b],
                                    out_h.at[pl.ds(base_w + kk * EB, EB)])

                    @pl.when(kk + 2 < nch)
                    def _():
                        issue(kk + 2, b)

            return 0

        lax.fori_loop(0, npair, pair, 0)

    return k(pd, ps, src, dst)


def _sc_scatter_add(msg, dst, zeros):
    e, d = msg.shape
    n = zeros.shape[0]
    ew = e // NW
    nch = ew // EB
    rps = (n // NS) // 8 * 8
    tail = n - NS * rps

    @functools.partial(
        pl.kernel,
        out_type=jax.ShapeDtypeStruct((NC, n, d), jnp.float32),
        mesh=_sc_mesh(),
        scratch_types=[
            pltpu.VMEM((EB,), jnp.int32),
            pltpu.VMEM((EB,), jnp.int32),
            pltpu.VMEM((2, EB, d), jnp.float32),
            pltpu.VMEM_SHARED((n, d), jnp.float32),
            pltpu.SemaphoreType.DMA,
            pltpu.SemaphoreType.DMA,
            pltpu.SemaphoreType.DMA,
            pltpu.SemaphoreType.DMA,
        ],
    )
    def k(msg_h, dst_h, z_h, out_h, idx0, idx1, mv, shared,
          si0, si1, sm0, sm1):
        cid = lax.axis_index("c")
        sid = lax.axis_index("s")
        wid = cid * NS + sid
        base_w = wid * ew
        idxs = (idx0, idx1)
        sems_i = (si0, si1)
        sems_m = (sm0, sm1)
        row0 = sid * rps
        pltpu.sync_copy(z_h.at[pl.ds(row0, rps)], shared.at[pl.ds(row0, rps)])

        @pl.when(sid == 0)
        def _():
            pltpu.sync_copy(z_h.at[pl.ds(NS * rps, tail)],
                            shared.at[pl.ds(NS * rps, tail)])

        plsc.subcore_barrier()

        def issue(kk, b):
            base = base_w + kk * EB
            pltpu.async_copy(dst_h.at[pl.ds(base, EB)], idxs[b], sems_i[b])
            pltpu.async_copy(msg_h.at[pl.ds(base, EB)], mv.at[b], sems_m[b])

        issue(0, 0)
        issue(1, 1)
        npair = (nch + 1) // 2

        def pair(p, _):
            for b in range(2):
                kk = 2 * p + b

                @pl.when(kk < nch)
                def _():
                    pltpu.make_async_copy(dst_h.at[pl.ds(base_w, EB)],
                                          idxs[b], sems_i[b]).wait()
                    pltpu.make_async_copy(msg_h.at[pl.ds(base_w, EB)],
                                          mv.at[b], sems_m[b]).wait()
                    pltpu.sync_copy(mv.at[b], shared.at[idxs[b]], add=True)

                    @pl.when(kk + 2 < nch)
                    def _():
                        issue(kk + 2, b)

            return 0

        lax.fori_loop(0, npair, pair, 0)
        plsc.subcore_barrier()
        pltpu.sync_copy(shared.at[pl.ds(row0, rps)],
                        out_h.at[cid, pl.ds(row0, rps)])

        @pl.when(sid == 0)
        def _():
            pltpu.sync_copy(shared.at[pl.ds(NS * rps, tail)],
                            out_h.at[cid, pl.ds(NS * rps, tail)])

    return k(msg, dst, zeros)


def kernel(x, edge_index, edge_attr, batch, atom_W, atom_b, edge_W, edge_b,
           Wf, bf, Ws, bs, gamma, beta, fc_hW, fc_hb, hid_W, hid_b,
           out_W, out_b):
    n, d = x.shape
    e = edge_attr.shape[0]
    src = edge_index[0]
    dst = edge_index[1]

    wds, wss, ms, cs = [], [], [], []
    for l in range(3):
        wf_i, wf_j, wf_e = Wf[l][:d], Wf[l][d:2 * d], Wf[l][2 * d:]
        ws_i, ws_j, ws_e = Ws[l][:d], Ws[l][d:2 * d], Ws[l][2 * d:]
        wds.append(jnp.concatenate([wf_i, ws_i], axis=1))
        wss.append(jnp.concatenate([wf_j, ws_j], axis=1))
        ms.append(jnp.concatenate([edge_W @ wf_e, edge_W @ ws_e], axis=1))
        cs.append(jnp.concatenate([edge_b @ wf_e + bf[l],
                                   edge_b @ ws_e + bs[l]]))
    mcat = jnp.concatenate(ms, axis=1)
    ccat = jnp.concatenate(cs)[None, :]

    h = _tc_mm_bias(x, atom_W, atom_b[None, :], block_rows=2000)
    c_layers = _tc_edge_const(edge_attr, mcat, ccat, block_rows=4000)
    zeros = jnp.zeros((n, d), dtype=jnp.float32)

    for l in range(3):
        pd, ps = _tc_proj(h, wds[l], wss[l], block_rows=2000)
        t = _sc_gather_sum(pd, ps, src, dst)
        msg = _tc_act(t, c_layers[l], block_rows=4000)
        aggp = _sc_scatter_add(msg, dst, zeros)
        h = _tc_norm(aggp, h, gamma[l][None, :], beta[l][None, :])

    out = _tc_pool_mlp(h, batch[:, None], fc_hW, fc_hb[None, :],
                       hid_W, hid_b, out_W, out_b[None, :])
    return jnp.reshape(out, (16,))


# speedup vs baseline: 3.3121x; 1.3235x over previous
import functools

import jax
import jax.numpy as jnp
from jax import lax
from jax.experimental import pallas as pl
from jax.experimental.pallas import tpu as pltpu
from jax.experimental.pallas import tpu_sc as plsc

NC = 2
NS = 16
NW = NC * NS
EB = 80


def _sc_mesh():
    return plsc.VectorSubcoreMesh(
        core_axis_name="c", subcore_axis_name="s", num_cores=NC, num_subcores=NS)


def _mm_bias_body(x_ref, w_ref, b_ref, o_ref):
    o_ref[...] = jnp.dot(x_ref[...], w_ref[...],
                         preferred_element_type=jnp.float32) + b_ref[...]


def _tc_mm_bias(x, w, b, block_rows):
    n, k = x.shape
    m = w.shape[1]
    grid = n // block_rows
    return pl.pallas_call(
        _mm_bias_body,
        grid=(grid,),
        in_specs=[
            pl.BlockSpec((block_rows, k), lambda i: (i, 0)),
            pl.BlockSpec((k, m), lambda i: (0, 0)),
            pl.BlockSpec((1, m), lambda i: (0, 0)),
        ],
        out_specs=pl.BlockSpec((block_rows, m), lambda i: (i, 0)),
        out_shape=jax.ShapeDtypeStruct((n, m), jnp.float32),
    )(x, w, b)


def _proj_body(h_ref, wd_ref, ws_ref, pd_ref, ps_ref):
    h = h_ref[...]
    pd_ref[...] = jnp.dot(h, wd_ref[...], preferred_element_type=jnp.float32)
    ps_ref[...] = jnp.dot(h, ws_ref[...], preferred_element_type=jnp.float32)


def _tc_proj(h, wd, ws, block_rows):
    n = h.shape[0]
    grid = n // block_rows
    out = jax.ShapeDtypeStruct((n, 256), jnp.float32)
    return pl.pallas_call(
        _proj_body,
        grid=(grid,),
        in_specs=[
            pl.BlockSpec((block_rows, 128), lambda i: (i, 0)),
            pl.BlockSpec((128, 256), lambda i: (0, 0)),
            pl.BlockSpec((128, 256), lambda i: (0, 0)),
        ],
        out_specs=[pl.BlockSpec((block_rows, 256), lambda i: (i, 0))] * 2,
        out_shape=[out, out],
    )(h, wd, ws)


def _act_body(t_ref, ea_ref, m_ref, c_ref, o_ref):
    t = (t_ref[...] + c_ref[...]
         + jnp.dot(ea_ref[...], m_ref[...], preferred_element_type=jnp.float32))
    o_ref[...] = jax.nn.sigmoid(t[:, :128]) * jax.nn.softplus(t[:, 128:])


def _tc_act(t, edge_attr, m, c, block_rows):
    e = t.shape[0]
    grid = e // block_rows
    return pl.pallas_call(
        _act_body,
        grid=(grid,),
        in_specs=[
            pl.BlockSpec((block_rows, 256), lambda i: (i, 0)),
            pl.BlockSpec((block_rows, 16), lambda i: (i, 0)),
            pl.BlockSpec((16, 256), lambda i: (0, 0)),
            pl.BlockSpec((1, 256), lambda i: (0, 0)),
        ],
        out_specs=pl.BlockSpec((block_rows, 128), lambda i: (i, 0)),
        out_shape=jax.ShapeDtypeStruct((e, 128), jnp.float32),
    )(t, edge_attr, m, c)


def _norm_body(aggp_ref, h_ref, g_ref, b_ref, o_ref):
    agg = aggp_ref[0] + aggp_ref[1]
    m = jnp.mean(agg, axis=0, keepdims=True)
    d = agg - m
    v = jnp.mean(d * d, axis=0, keepdims=True)
    o_ref[...] = d * lax.rsqrt(v + 1e-5) * g_ref[...] + b_ref[...] + h_ref[...]


def _tc_norm(aggp, h, g, b):
    return pl.pallas_call(
        _norm_body,
        out_shape=jax.ShapeDtypeStruct(h.shape, jnp.float32),
    )(aggp, h, g, b)


def _pool_mlp_body(h_ref, batch_ref, fcw_ref, fcb_ref, hw_ref, hb_ref,
                   ow_ref, ob_ref, o_ref):
    n = h_ref.shape[0]
    gidx = lax.broadcasted_iota(jnp.int32, (n, 16), 1)
    oh = (batch_ref[...] == gidx).astype(jnp.float32)
    sums = lax.dot_general(oh, h_ref[...], (((0,), (0,)), ((), ())),
                           preferred_element_type=jnp.float32)
    cnts = jnp.sum(oh, axis=0)
    pooled = sums / jnp.clip(cnts, 1.0)[:, None]
    h2 = jnp.dot(pooled, fcw_ref[...],
                 preferred_element_type=jnp.float32) + fcb_ref[...]
    for l in range(3):
        h2 = jax.nn.relu(jnp.dot(h2, hw_ref[l],
                                 preferred_element_type=jnp.float32) + hb_ref[l])
    out = jnp.dot(h2, ow_ref[...], preferred_element_type=jnp.float32)
    o_ref[...] = out.T + ob_ref[...]


def _tc_pool_mlp(h, batch2d, fc_hW, fc_hb, hid_W, hid_b, out_W, out_b):
    return pl.pallas_call(
        _pool_mlp_body,
        out_shape=jax.ShapeDtypeStruct((1, 16), jnp.float32),
    )(h, batch2d, fc_hW, fc_hb, hid_W, hid_b, out_W, out_b)


GEB = 40
GNB = 3


def _sc_gather_sum(pd, ps, src, dst):
    n = pd.shape[0]
    e = src.shape[0]
    ew = e // NW
    nch = ew // GEB
    ngrp = (nch + GNB - 1) // GNB

    @functools.partial(
        pl.kernel,
        out_type=jax.ShapeDtypeStruct((e, 256), jnp.float32),
        mesh=_sc_mesh(),
        scratch_types=[
            pltpu.VMEM((ew,), jnp.int32),
            pltpu.VMEM((ew,), jnp.int32),
            pltpu.VMEM((GNB, GEB, 256), jnp.float32),
            pltpu.VMEM((GNB, GEB, 256), jnp.float32),
            pltpu.VMEM((GNB, GEB, 256), jnp.float32),
        ] + [pltpu.SemaphoreType.DMA] * (3 * GNB),
    )
    def k(pd_h, ps_h, src_h, dst_h, out_h, idxd, idxs, rd, rs, ts, *sems):
        sems_d = sems[0:GNB]
        sems_s = sems[GNB:2 * GNB]
        sems_w = sems[2 * GNB:3 * GNB]
        wid = lax.axis_index("c") * NS + lax.axis_index("s")
        base_w = wid * ew
        pltpu.sync_copy(dst_h.at[pl.ds(base_w, ew)], idxd)
        pltpu.sync_copy(src_h.at[pl.ds(base_w, ew)], idxs)

        def issue(kk, b):
            isl = pl.ds(kk * GEB, GEB)
            pltpu.async_copy(pd_h.at[idxd.at[isl]], rd.at[b], sems_d[b])
            pltpu.async_copy(ps_h.at[idxs.at[isl]], rs.at[b], sems_s[b])

        for b in range(GNB):
            issue(b, b)

        def grp(p, _):
            for b in range(GNB):
                kk = GNB * p + b

                @pl.when(kk < nch)
                def _():
                    pltpu.make_async_copy(pd_h.at[idxd.at[pl.ds(0, GEB)]],
                                          rd.at[b], sems_d[b]).wait()
                    pltpu.make_async_copy(ps_h.at[idxs.at[pl.ds(0, GEB)]],
                                          rs.at[b], sems_s[b]).wait()

                    def ebody(i, _):
                        for j in range(16):
                            sl = pl.ds(j * 16, 16)
                            ts[b, i, sl] = rd[b, i, sl] + rs[b, i, sl]
                        return 0

                    lax.fori_loop(0, GEB, ebody, 0)

                    @pl.when(kk + GNB < nch)
                    def _():
                        issue(kk + GNB, b)

                    @pl.when(kk >= GNB)
                    def _():
                        pltpu.make_async_copy(
                            ts.at[b], out_h.at[pl.ds(base_w, GEB)],
                            sems_w[b]).wait()

                    pltpu.async_copy(ts.at[b],
                                     out_h.at[pl.ds(base_w + kk * GEB, GEB)],
                                     sems_w[b])

            return 0

        lax.fori_loop(0, ngrp, grp, 0)
        for b in range(GNB):
            pltpu.make_async_copy(ts.at[b], out_h.at[pl.ds(base_w, GEB)],
                                  sems_w[b]).wait()

    return k(pd, ps, src, dst)


def _sc_scatter_add(msg, dst, zeros):
    e, d = msg.shape
    n = zeros.shape[0]
    ew = e // NW
    nch = ew // EB
    rps = (n // NS) // 8 * 8
    tail = n - NS * rps

    @functools.partial(
        pl.kernel,
        out_type=jax.ShapeDtypeStruct((NC, n, d), jnp.float32),
        mesh=_sc_mesh(),
        scratch_types=[
            pltpu.VMEM((EB,), jnp.int32),
            pltpu.VMEM((EB,), jnp.int32),
            pltpu.VMEM((2, EB, d), jnp.float32),
            pltpu.VMEM_SHARED((n, d), jnp.float32),
            pltpu.SemaphoreType.DMA,
            pltpu.SemaphoreType.DMA,
            pltpu.SemaphoreType.DMA,
            pltpu.SemaphoreType.DMA,
        ],
    )
    def k(msg_h, dst_h, z_h, out_h, idx0, idx1, mv, shared,
          si0, si1, sm0, sm1):
        cid = lax.axis_index("c")
        sid = lax.axis_index("s")
        wid = cid * NS + sid
        base_w = wid * ew
        idxs = (idx0, idx1)
        sems_i = (si0, si1)
        sems_m = (sm0, sm1)
        row0 = sid * rps
        pltpu.sync_copy(z_h.at[pl.ds(row0, rps)], shared.at[pl.ds(row0, rps)])

        @pl.when(sid == 0)
        def _():
            pltpu.sync_copy(z_h.at[pl.ds(NS * rps, tail)],
                            shared.at[pl.ds(NS * rps, tail)])

        plsc.subcore_barrier()

        def issue(kk, b):
            base = base_w + kk * EB
            pltpu.async_copy(dst_h.at[pl.ds(base, EB)], idxs[b], sems_i[b])
            pltpu.async_copy(msg_h.at[pl.ds(base, EB)], mv.at[b], sems_m[b])

        issue(0, 0)
        issue(1, 1)
        npair = (nch + 1) // 2

        def pair(p, _):
            for b in range(2):
                kk = 2 * p + b

                @pl.when(kk < nch)
                def _():
                    pltpu.make_async_copy(dst_h.at[pl.ds(base_w, EB)],
                                          idxs[b], sems_i[b]).wait()
                    pltpu.make_async_copy(msg_h.at[pl.ds(base_w, EB)],
                                          mv.at[b], sems_m[b]).wait()
                    pltpu.sync_copy(mv.at[b], shared.at[idxs[b]], add=True)

                    @pl.when(kk + 2 < nch)
                    def _():
                        issue(kk + 2, b)

            return 0

        lax.fori_loop(0, npair, pair, 0)
        plsc.subcore_barrier()
        pltpu.sync_copy(shared.at[pl.ds(row0, rps)],
                        out_h.at[cid, pl.ds(row0, rps)])

        @pl.when(sid == 0)
        def _():
            pltpu.sync_copy(shared.at[pl.ds(NS * rps, tail)],
                            out_h.at[cid, pl.ds(NS * rps, tail)])

    return k(msg, dst, zeros)


def kernel(x, edge_index, edge_attr, batch, atom_W, atom_b, edge_W, edge_b,
           Wf, bf, Ws, bs, gamma, beta, fc_hW, fc_hb, hid_W, hid_b,
           out_W, out_b):
    n, d = x.shape
    e = edge_attr.shape[0]
    src = edge_index[0]
    dst = edge_index[1]

    wds, wss, ms, cs = [], [], [], []
    for l in range(3):
        wf_i, wf_j, wf_e = Wf[l][:d], Wf[l][d:2 * d], Wf[l][2 * d:]
        ws_i, ws_j, ws_e = Ws[l][:d], Ws[l][d:2 * d], Ws[l][2 * d:]
        wds.append(jnp.concatenate([wf_i, ws_i], axis=1))
        wss.append(jnp.concatenate([wf_j, ws_j], axis=1))
        ms.append(jnp.concatenate([edge_W @ wf_e, edge_W @ ws_e], axis=1))
        cs.append(jnp.concatenate([edge_b @ wf_e + bf[l],
                                   edge_b @ ws_e + bs[l]]))
    h = _tc_mm_bias(x, atom_W, atom_b[None, :], block_rows=2000)
    zeros = jnp.zeros((n, d), dtype=jnp.float32)

    for l in range(3):
        pd, ps = _tc_proj(h, wds[l], wss[l], block_rows=2000)
        t = _sc_gather_sum(pd, ps, src, dst)
        msg = _tc_act(t, edge_attr, ms[l], cs[l][None, :], block_rows=4000)
        aggp = _sc_scatter_add(msg, dst, zeros)
        h = _tc_norm(aggp, h, gamma[l][None, :], beta[l][None, :])

    out = _tc_pool_mlp(h, batch[:, None], fc_hW, fc_hb[None, :],
                       hid_W, hid_b, out_W, out_b[None, :])
    return jnp.reshape(out, (16,))


# speedup vs baseline: 3.4557x; 1.0433x over previous
import functools

import jax
import jax.numpy as jnp
from jax import lax
from jax.experimental import pallas as pl
from jax.experimental.pallas import tpu as pltpu
from jax.experimental.pallas import tpu_sc as plsc

NC = 2
NS = 16
NW = NC * NS
EB = 80


def _sc_mesh():
    return plsc.VectorSubcoreMesh(
        core_axis_name="c", subcore_axis_name="s", num_cores=NC, num_subcores=NS)


def _mm_bias_body(x_ref, w_ref, b_ref, o_ref):
    o_ref[...] = jnp.dot(x_ref[...], w_ref[...],
                         preferred_element_type=jnp.float32) + b_ref[...]


def _tc_mm_bias(x, w, b, block_rows):
    n, k = x.shape
    m = w.shape[1]
    grid = n // block_rows
    return pl.pallas_call(
        _mm_bias_body,
        grid=(grid,),
        in_specs=[
            pl.BlockSpec((block_rows, k), lambda i: (i, 0)),
            pl.BlockSpec((k, m), lambda i: (0, 0)),
            pl.BlockSpec((1, m), lambda i: (0, 0)),
        ],
        out_specs=pl.BlockSpec((block_rows, m), lambda i: (i, 0)),
        out_shape=jax.ShapeDtypeStruct((n, m), jnp.float32),
    )(x, w, b)


def _proj_body(h_ref, wd_ref, ws_ref, pd_ref, ps_ref):
    h = h_ref[...]
    pd_ref[...] = jnp.dot(h, wd_ref[...], preferred_element_type=jnp.float32)
    ps_ref[...] = jnp.dot(h, ws_ref[...], preferred_element_type=jnp.float32)


def _tc_proj(h, wd, ws, block_rows):
    n = h.shape[0]
    grid = n // block_rows
    out = jax.ShapeDtypeStruct((n, 256), jnp.float32)
    return pl.pallas_call(
        _proj_body,
        grid=(grid,),
        in_specs=[
            pl.BlockSpec((block_rows, 128), lambda i: (i, 0)),
            pl.BlockSpec((128, 256), lambda i: (0, 0)),
            pl.BlockSpec((128, 256), lambda i: (0, 0)),
        ],
        out_specs=[pl.BlockSpec((block_rows, 256), lambda i: (i, 0))] * 2,
        out_shape=[out, out],
    )(h, wd, ws)


def _act_body(t_ref, ea_ref, m_ref, c_ref, o_ref):
    t = (t_ref[...].astype(jnp.float32) + c_ref[...]
         + jnp.dot(ea_ref[...], m_ref[...], preferred_element_type=jnp.float32))
    o_ref[...] = jax.nn.sigmoid(t[:, :128]) * jax.nn.softplus(t[:, 128:])


def _tc_act(t, edge_attr, m, c, block_rows):
    e = t.shape[0]
    grid = e // block_rows
    return pl.pallas_call(
        _act_body,
        grid=(grid,),
        in_specs=[
            pl.BlockSpec((block_rows, 256), lambda i: (i, 0)),
            pl.BlockSpec((block_rows, 16), lambda i: (i, 0)),
            pl.BlockSpec((16, 256), lambda i: (0, 0)),
            pl.BlockSpec((1, 256), lambda i: (0, 0)),
        ],
        out_specs=pl.BlockSpec((block_rows, 128), lambda i: (i, 0)),
        out_shape=jax.ShapeDtypeStruct((e, 128), jnp.float32),
    )(t, edge_attr, m, c)


def _norm_body(aggp_ref, h_ref, g_ref, b_ref, o_ref):
    agg = aggp_ref[0] + aggp_ref[1]
    m = jnp.mean(agg, axis=0, keepdims=True)
    d = agg - m
    v = jnp.mean(d * d, axis=0, keepdims=True)
    o_ref[...] = d * lax.rsqrt(v + 1e-5) * g_ref[...] + b_ref[...] + h_ref[...]


def _tc_norm(aggp, h, g, b):
    return pl.pallas_call(
        _norm_body,
        out_shape=jax.ShapeDtypeStruct(h.shape, jnp.float32),
    )(aggp, h, g, b)


def _pool_mlp_body(h_ref, batch_ref, fcw_ref, fcb_ref, hw_ref, hb_ref,
                   ow_ref, ob_ref, o_ref):
    n = h_ref.shape[0]
    gidx = lax.broadcasted_iota(jnp.int32, (n, 16), 1)
    oh = (batch_ref[...] == gidx).astype(jnp.float32)
    sums = lax.dot_general(oh, h_ref[...], (((0,), (0,)), ((), ())),
                           preferred_element_type=jnp.float32)
    cnts = jnp.sum(oh, axis=0)
    pooled = sums / jnp.clip(cnts, 1.0)[:, None]
    h2 = jnp.dot(pooled, fcw_ref[...],
                 preferred_element_type=jnp.float32) + fcb_ref[...]
    for l in range(3):
        h2 = jax.nn.relu(jnp.dot(h2, hw_ref[l],
                                 preferred_element_type=jnp.float32) + hb_ref[l])
    out = jnp.dot(h2, ow_ref[...], preferred_element_type=jnp.float32)
    o_ref[...] = out.T + ob_ref[...]


def _tc_pool_mlp(h, batch2d, fc_hW, fc_hb, hid_W, hid_b, out_W, out_b):
    return pl.pallas_call(
        _pool_mlp_body,
        out_shape=jax.ShapeDtypeStruct((1, 16), jnp.float32),
    )(h, batch2d, fc_hW, fc_hb, hid_W, hid_b, out_W, out_b)


GEB = 40
GNB = 3


def _sc_gather_sum(pd, ps, src, dst):
    n = pd.shape[0]
    e = src.shape[0]
    ew = e // NW
    nch = ew // GEB
    ngrp = (nch + GNB - 1) // GNB

    @functools.partial(
        pl.kernel,
        out_type=jax.ShapeDtypeStruct((e, 256), jnp.float32),
        mesh=_sc_mesh(),
        scratch_types=[
            pltpu.VMEM((ew,), jnp.int32),
            pltpu.VMEM((ew,), jnp.int32),
            pltpu.VMEM((GNB, GEB, 256), jnp.float32),
            pltpu.VMEM((GNB, GEB, 256), jnp.float32),
            pltpu.VMEM((GNB, GEB, 256), jnp.float32),
        ] + [pltpu.SemaphoreType.DMA] * (3 * GNB),
    )
    def k(pd_h, ps_h, src_h, dst_h, out_h, idxd, idxs, rd, rs, ts, *sems):
        sems_d = sems[0:GNB]
        sems_s = sems[GNB:2 * GNB]
        sems_w = sems[2 * GNB:3 * GNB]
        wid = lax.axis_index("c") * NS + lax.axis_index("s")
        base_w = wid * ew
        pltpu.sync_copy(dst_h.at[pl.ds(base_w, ew)], idxd)
        pltpu.sync_copy(src_h.at[pl.ds(base_w, ew)], idxs)

        def issue(kk, b):
            isl = pl.ds(kk * GEB, GEB)
            pltpu.async_copy(pd_h.at[idxd.at[isl]], rd.at[b], sems_d[b])
            pltpu.async_copy(ps_h.at[idxs.at[isl]], rs.at[b], sems_s[b])

        for b in range(GNB):
            issue(b, b)

        def grp(p, _):
            for b in range(GNB):
                kk = GNB * p + b

                @pl.when(kk < nch)
                def _():
                    pltpu.make_async_copy(pd_h.at[idxd.at[pl.ds(0, GEB)]],
                                          rd.at[b], sems_d[b]).wait()
                    pltpu.make_async_copy(ps_h.at[idxs.at[pl.ds(0, GEB)]],
                                          rs.at[b], sems_s[b]).wait()

                    def ebody(i, _):
                        for j in range(16):
                            sl = pl.ds(j * 16, 16)
                            ts[b, i, sl] = rd[b, i, sl] + rs[b, i, sl]
                        return 0

                    lax.fori_loop(0, GEB, ebody, 0, unroll=8)

                    @pl.when(kk + GNB < nch)
                    def _():
                        issue(kk + GNB, b)

                    @pl.when(kk >= GNB)
                    def _():
                        pltpu.make_async_copy(
                            ts.at[b], out_h.at[pl.ds(base_w, GEB)],
                            sems_w[b]).wait()

                    pltpu.async_copy(ts.at[b],
                                     out_h.at[pl.ds(base_w + kk * GEB, GEB)],
                                     sems_w[b])

            return 0

        lax.fori_loop(0, ngrp, grp, 0)
        for b in range(GNB):
            pltpu.make_async_copy(ts.at[b], out_h.at[pl.ds(base_w, GEB)],
                                  sems_w[b]).wait()

    return k(pd, ps, src, dst)


def _sc_scatter_add(msg, dst, zeros):
    e, d = msg.shape
    n = zeros.shape[0]
    ew = e // NW
    nch = ew // EB
    rps = (n // NS) // 8 * 8
    tail = n - NS * rps

    @functools.partial(
        pl.kernel,
        out_type=jax.ShapeDtypeStruct((NC, n, d), jnp.float32),
        mesh=_sc_mesh(),
        scratch_types=[
            pltpu.VMEM((EB,), jnp.int32),
            pltpu.VMEM((EB,), jnp.int32),
            pltpu.VMEM((2, EB, d), jnp.float32),
            pltpu.VMEM_SHARED((n, d), jnp.float32),
            pltpu.SemaphoreType.DMA,
            pltpu.SemaphoreType.DMA,
            pltpu.SemaphoreType.DMA,
            pltpu.SemaphoreType.DMA,
        ],
    )
    def k(msg_h, dst_h, z_h, out_h, idx0, idx1, mv, shared,
          si0, si1, sm0, sm1):
        cid = lax.axis_index("c")
        sid = lax.axis_index("s")
        wid = cid * NS + sid
        base_w = wid * ew
        idxs = (idx0, idx1)
        sems_i = (si0, si1)
        sems_m = (sm0, sm1)
        row0 = sid * rps
        pltpu.sync_copy(z_h.at[pl.ds(row0, rps)], shared.at[pl.ds(row0, rps)])

        @pl.when(sid == 0)
        def _():
            pltpu.sync_copy(z_h.at[pl.ds(NS * rps, tail)],
                            shared.at[pl.ds(NS * rps, tail)])

        plsc.subcore_barrier()

        def issue(kk, b):
            base = base_w + kk * EB
            pltpu.async_copy(dst_h.at[pl.ds(base, EB)], idxs[b], sems_i[b])
            pltpu.async_copy(msg_h.at[pl.ds(base, EB)], mv.at[b], sems_m[b])

        issue(0, 0)
        issue(1, 1)
        npair = (nch + 1) // 2

        def pair(p, _):
            for b in range(2):
                kk = 2 * p + b

                @pl.when(kk < nch)
                def _():
                    pltpu.make_async_copy(dst_h.at[pl.ds(base_w, EB)],
                                          idxs[b], sems_i[b]).wait()
                    pltpu.make_async_copy(msg_h.at[pl.ds(base_w, EB)],
                                          mv.at[b], sems_m[b]).wait()
                    pltpu.sync_copy(mv.at[b], shared.at[idxs[b]], add=True)

                    @pl.when(kk + 2 < nch)
                    def _():
                        issue(kk + 2, b)

            return 0

        lax.fori_loop(0, npair, pair, 0)
        plsc.subcore_barrier()
        pltpu.sync_copy(shared.at[pl.ds(row0, rps)],
                        out_h.at[cid, pl.ds(row0, rps)])

        @pl.when(sid == 0)
        def _():
            pltpu.sync_copy(shared.at[pl.ds(NS * rps, tail)],
                            out_h.at[cid, pl.ds(NS * rps, tail)])

    return k(msg, dst, zeros)


def kernel(x, edge_index, edge_attr, batch, atom_W, atom_b, edge_W, edge_b,
           Wf, bf, Ws, bs, gamma, beta, fc_hW, fc_hb, hid_W, hid_b,
           out_W, out_b):
    n, d = x.shape
    e = edge_attr.shape[0]
    src = edge_index[0]
    dst = edge_index[1]

    wds, wss, ms, cs = [], [], [], []
    for l in range(3):
        wf_i, wf_j, wf_e = Wf[l][:d], Wf[l][d:2 * d], Wf[l][2 * d:]
        ws_i, ws_j, ws_e = Ws[l][:d], Ws[l][d:2 * d], Ws[l][2 * d:]
        wds.append(jnp.concatenate([wf_i, ws_i], axis=1))
        wss.append(jnp.concatenate([wf_j, ws_j], axis=1))
        ms.append(jnp.concatenate([edge_W @ wf_e, edge_W @ ws_e], axis=1))
        cs.append(jnp.concatenate([edge_b @ wf_e + bf[l],
                                   edge_b @ ws_e + bs[l]]))
    h = _tc_mm_bias(x, atom_W, atom_b[None, :], block_rows=2000)
    zeros = jnp.zeros((n, d), dtype=jnp.float32)

    for l in range(3):
        pd, ps = _tc_proj(h, wds[l], wss[l], block_rows=2000)
        t = _sc_gather_sum(pd, ps, src, dst)
        msg = _tc_act(t, edge_attr, ms[l], cs[l][None, :], block_rows=4000)
        aggp = _sc_scatter_add(msg, dst, zeros)
        h = _tc_norm(aggp, h, gamma[l][None, :], beta[l][None, :])

    out = _tc_pool_mlp(h, batch[:, None], fc_hW, fc_hb[None, :],
                       hid_W, hid_b, out_W, out_b[None, :])
    return jnp.reshape(out, (16,))


# speedup vs baseline: 3.5723x; 1.0337x over previous
import functools

import jax
import jax.numpy as jnp
from jax import lax
from jax.experimental import pallas as pl
from jax.experimental.pallas import tpu as pltpu
from jax.experimental.pallas import tpu_sc as plsc

NC = 2
NS = 16
NW = NC * NS
EB = 80


def _sc_mesh():
    return plsc.VectorSubcoreMesh(
        core_axis_name="c", subcore_axis_name="s", num_cores=NC, num_subcores=NS)


def _mm_bias_body(x_ref, w_ref, b_ref, o_ref):
    o_ref[...] = jnp.dot(x_ref[...], w_ref[...],
                         preferred_element_type=jnp.float32) + b_ref[...]


def _tc_mm_bias(x, w, b, block_rows):
    n, k = x.shape
    m = w.shape[1]
    grid = n // block_rows
    return pl.pallas_call(
        _mm_bias_body,
        grid=(grid,),
        in_specs=[
            pl.BlockSpec((block_rows, k), lambda i: (i, 0)),
            pl.BlockSpec((k, m), lambda i: (0, 0)),
            pl.BlockSpec((1, m), lambda i: (0, 0)),
        ],
        out_specs=pl.BlockSpec((block_rows, m), lambda i: (i, 0)),
        out_shape=jax.ShapeDtypeStruct((n, m), jnp.float32),
    )(x, w, b)


def _proj_body(h_ref, wd_ref, ws_ref, pd_ref, ps_ref):
    h = h_ref[...]
    pd_ref[...] = jnp.dot(h, wd_ref[...], preferred_element_type=jnp.float32)
    ps_ref[...] = jnp.dot(h, ws_ref[...], preferred_element_type=jnp.float32)


def _tc_proj(h, wd, ws, block_rows):
    n = h.shape[0]
    grid = n // block_rows
    out = jax.ShapeDtypeStruct((n, 256), jnp.float32)
    return pl.pallas_call(
        _proj_body,
        grid=(grid,),
        in_specs=[
            pl.BlockSpec((block_rows, 128), lambda i: (i, 0)),
            pl.BlockSpec((128, 256), lambda i: (0, 0)),
            pl.BlockSpec((128, 256), lambda i: (0, 0)),
        ],
        out_specs=[pl.BlockSpec((block_rows, 256), lambda i: (i, 0))] * 2,
        out_shape=[out, out],
    )(h, wd, ws)


def _act_body(t_ref, ea_ref, m_ref, c_ref, o_ref):
    t = (t_ref[...].astype(jnp.float32) + c_ref[...]
         + jnp.dot(ea_ref[...], m_ref[...], preferred_element_type=jnp.float32))
    o_ref[...] = jax.nn.sigmoid(t[:, :128]) * jax.nn.softplus(t[:, 128:])


def _tc_act(t, edge_attr, m, c, block_rows):
    e = t.shape[0]
    grid = e // block_rows
    return pl.pallas_call(
        _act_body,
        grid=(grid,),
        in_specs=[
            pl.BlockSpec((block_rows, 256), lambda i: (i, 0)),
            pl.BlockSpec((block_rows, 16), lambda i: (i, 0)),
            pl.BlockSpec((16, 256), lambda i: (0, 0)),
            pl.BlockSpec((1, 256), lambda i: (0, 0)),
        ],
        out_specs=pl.BlockSpec((block_rows, 128), lambda i: (i, 0)),
        out_shape=jax.ShapeDtypeStruct((e, 128), jnp.float32),
    )(t, edge_attr, m, c)


def _norm_body(agg0_ref, agg1_ref, h_ref, g_ref, b_ref, o_ref):
    agg = (agg0_ref[0] + agg0_ref[1]) + (agg1_ref[0] + agg1_ref[1])
    m = jnp.mean(agg, axis=0, keepdims=True)
    d = agg - m
    v = jnp.mean(d * d, axis=0, keepdims=True)
    o_ref[...] = d * lax.rsqrt(v + 1e-5) * g_ref[...] + b_ref[...] + h_ref[...]


def _tc_norm(aggp0, aggp1, h, g, b):
    return pl.pallas_call(
        _norm_body,
        out_shape=jax.ShapeDtypeStruct(h.shape, jnp.float32),
    )(aggp0, aggp1, h, g, b)


def _pool_mlp_body(h_ref, batch_ref, fcw_ref, fcb_ref, hw_ref, hb_ref,
                   ow_ref, ob_ref, o_ref):
    n = h_ref.shape[0]
    gidx = lax.broadcasted_iota(jnp.int32, (n, 16), 1)
    oh = (batch_ref[...] == gidx).astype(jnp.float32)
    sums = lax.dot_general(oh, h_ref[...], (((0,), (0,)), ((), ())),
                           preferred_element_type=jnp.float32)
    cnts = jnp.sum(oh, axis=0)
    pooled = sums / jnp.clip(cnts, 1.0)[:, None]
    h2 = jnp.dot(pooled, fcw_ref[...],
                 preferred_element_type=jnp.float32) + fcb_ref[...]
    for l in range(3):
        h2 = jax.nn.relu(jnp.dot(h2, hw_ref[l],
                                 preferred_element_type=jnp.float32) + hb_ref[l])
    out = jnp.dot(h2, ow_ref[...], preferred_element_type=jnp.float32)
    o_ref[...] = out.T + ob_ref[...]


def _tc_pool_mlp(h, batch2d, fc_hW, fc_hb, hid_W, hid_b, out_W, out_b):
    return pl.pallas_call(
        _pool_mlp_body,
        out_shape=jax.ShapeDtypeStruct((1, 16), jnp.float32),
    )(h, batch2d, fc_hW, fc_hb, hid_W, hid_b, out_W, out_b)


GEB = 40
GNB = 3


def _sc_gather_sum(pd, ps, src, dst):
    n = pd.shape[0]
    e = src.shape[0]
    ew = e // NW
    nch = ew // GEB
    ngrp = (nch + GNB - 1) // GNB

    @functools.partial(
        pl.kernel,
        out_type=jax.ShapeDtypeStruct((e, 256), jnp.float32),
        mesh=_sc_mesh(),
        scratch_types=[
            pltpu.VMEM((ew,), jnp.int32),
            pltpu.VMEM((ew,), jnp.int32),
            pltpu.VMEM((GNB, GEB, 256), jnp.float32),
            pltpu.VMEM((GNB, GEB, 256), jnp.float32),
            pltpu.VMEM((GNB, GEB, 256), jnp.float32),
        ] + [pltpu.SemaphoreType.DMA] * (3 * GNB),
    )
    def k(pd_h, ps_h, src_h, dst_h, out_h, idxd, idxs, rd, rs, ts, *sems):
        sems_d = sems[0:GNB]
        sems_s = sems[GNB:2 * GNB]
        sems_w = sems[2 * GNB:3 * GNB]
        wid = lax.axis_index("c") * NS + lax.axis_index("s")
        base_w = wid * ew
        pltpu.sync_copy(dst_h.at[pl.ds(base_w, ew)], idxd)
        pltpu.sync_copy(src_h.at[pl.ds(base_w, ew)], idxs)

        def issue(kk, b):
            isl = pl.ds(kk * GEB, GEB)
            pltpu.async_copy(pd_h.at[idxd.at[isl]], rd.at[b], sems_d[b])
            pltpu.async_copy(ps_h.at[idxs.at[isl]], rs.at[b], sems_s[b])

        for b in range(GNB):
            issue(b, b)

        def grp(p, _):
            for b in range(GNB):
                kk = GNB * p + b

                @pl.when(kk < nch)
                def _():
                    pltpu.make_async_copy(pd_h.at[idxd.at[pl.ds(0, GEB)]],
                                          rd.at[b], sems_d[b]).wait()
                    pltpu.make_async_copy(ps_h.at[idxs.at[pl.ds(0, GEB)]],
                                          rs.at[b], sems_s[b]).wait()

                    def ebody(i, _):
                        for j in range(16):
                            sl = pl.ds(j * 16, 16)
                            ts[b, i, sl] = rd[b, i, sl] + rs[b, i, sl]
                        return 0

                    lax.fori_loop(0, GEB, ebody, 0, unroll=8)

                    @pl.when(kk + GNB < nch)
                    def _():
                        issue(kk + GNB, b)

                    @pl.when(kk >= GNB)
                    def _():
                        pltpu.make_async_copy(
                            ts.at[b], out_h.at[pl.ds(base_w, GEB)],
                            sems_w[b]).wait()

                    pltpu.async_copy(ts.at[b],
                                     out_h.at[pl.ds(base_w + kk * GEB, GEB)],
                                     sems_w[b])

            return 0

        lax.fori_loop(0, ngrp, grp, 0)
        for b in range(GNB):
            pltpu.make_async_copy(ts.at[b], out_h.at[pl.ds(base_w, GEB)],
                                  sems_w[b]).wait()

    return k(pd, ps, src, dst)


def _sc_scatter_add(msg, dst, zeros):
    e, d = msg.shape
    n = zeros.shape[0]
    ew = e // NW
    nch = ew // EB
    rps = (n // NS) // 8 * 8
    tail = n - NS * rps

    @functools.partial(
        pl.kernel,
        out_type=jax.ShapeDtypeStruct((NC, n, d), jnp.float32),
        mesh=_sc_mesh(),
        scratch_types=[
            pltpu.VMEM((EB,), jnp.int32),
            pltpu.VMEM((EB,), jnp.int32),
            pltpu.VMEM((2, EB, d), jnp.float32),
            pltpu.VMEM_SHARED((n, d), jnp.float32),
            pltpu.SemaphoreType.DMA,
            pltpu.SemaphoreType.DMA,
            pltpu.SemaphoreType.DMA,
            pltpu.SemaphoreType.DMA,
        ],
    )
    def k(msg_h, dst_h, z_h, out_h, idx0, idx1, mv, shared,
          si0, si1, sm0, sm1):
        cid = lax.axis_index("c")
        sid = lax.axis_index("s")
        wid = cid * NS + sid
        base_w = wid * ew
        idxs = (idx0, idx1)
        sems_i = (si0, si1)
        sems_m = (sm0, sm1)
        row0 = sid * rps
        pltpu.sync_copy(z_h.at[pl.ds(row0, rps)], shared.at[pl.ds(row0, rps)])

        @pl.when(sid == 0)
        def _():
            pltpu.sync_copy(z_h.at[pl.ds(NS * rps, tail)],
                            shared.at[pl.ds(NS * rps, tail)])

        plsc.subcore_barrier()

        def issue(kk, b):
            base = base_w + kk * EB
            pltpu.async_copy(dst_h.at[pl.ds(base, EB)], idxs[b], sems_i[b])
            pltpu.async_copy(msg_h.at[pl.ds(base, EB)], mv.at[b], sems_m[b])

        issue(0, 0)
        issue(1, 1)
        npair = (nch + 1) // 2

        def pair(p, _):
            for b in range(2):
                kk = 2 * p + b

                @pl.when(kk < nch)
                def _():
                    pltpu.make_async_copy(dst_h.at[pl.ds(base_w, EB)],
                                          idxs[b], sems_i[b]).wait()
                    pltpu.make_async_copy(msg_h.at[pl.ds(base_w, EB)],
                                          mv.at[b], sems_m[b]).wait()
                    pltpu.sync_copy(mv.at[b], shared.at[idxs[b]], add=True)

                    @pl.when(kk + 2 < nch)
                    def _():
                        issue(kk + 2, b)

            return 0

        lax.fori_loop(0, npair, pair, 0)
        plsc.subcore_barrier()
        pltpu.sync_copy(shared.at[pl.ds(row0, rps)],
                        out_h.at[cid, pl.ds(row0, rps)])

        @pl.when(sid == 0)
        def _():
            pltpu.sync_copy(shared.at[pl.ds(NS * rps, tail)],
                            out_h.at[cid, pl.ds(NS * rps, tail)])

    return k(msg, dst, zeros)


def kernel(x, edge_index, edge_attr, batch, atom_W, atom_b, edge_W, edge_b,
           Wf, bf, Ws, bs, gamma, beta, fc_hW, fc_hb, hid_W, hid_b,
           out_W, out_b):
    n, d = x.shape
    e = edge_attr.shape[0]
    src = edge_index[0]
    dst = edge_index[1]

    wds, wss, ms, cs = [], [], [], []
    for l in range(3):
        wf_i, wf_j, wf_e = Wf[l][:d], Wf[l][d:2 * d], Wf[l][2 * d:]
        ws_i, ws_j, ws_e = Ws[l][:d], Ws[l][d:2 * d], Ws[l][2 * d:]
        wds.append(jnp.concatenate([wf_i, ws_i], axis=1))
        wss.append(jnp.concatenate([wf_j, ws_j], axis=1))
        ms.append(jnp.concatenate([edge_W @ wf_e, edge_W @ ws_e], axis=1))
        cs.append(jnp.concatenate([edge_b @ wf_e + bf[l],
                                   edge_b @ ws_e + bs[l]]))
    h = _tc_mm_bias(x, atom_W, atom_b[None, :], block_rows=2000)
    zeros = jnp.zeros((n, d), dtype=jnp.float32)

    eh = e // 2
    src_h = (src[:eh], src[eh:])
    dst_h = (dst[:eh], dst[eh:])
    ea_h = (edge_attr[:eh], edge_attr[eh:])

    for l in range(3):
        pd, ps = _tc_proj(h, wds[l], wss[l], block_rows=2000)
        aggp = []
        for p in range(2):
            t = _sc_gather_sum(pd, ps, src_h[p], dst_h[p])
            msg = _tc_act(t, ea_h[p], ms[l], cs[l][None, :], block_rows=4000)
            aggp.append(_sc_scatter_add(msg, dst_h[p], zeros))
        h = _tc_norm(aggp[0], aggp[1], h, gamma[l][None, :], beta[l][None, :])

    out = _tc_pool_mlp(h, batch[:, None], fc_hW, fc_hb[None, :],
                       hid_W, hid_b, out_W, out_b[None, :])
    return jnp.reshape(out, (16,))
